# baseline (speedup 1.0000x reference)
import jax
import jax.numpy as jnp
from jax import lax
from jax.experimental import pallas as pl
from jax.experimental.pallas import tpu as pltpu
from jax.experimental.pallas import tpu_sc as plsc

_N = 10000
_E = 320000
_D = 128
_H = 8
_G = 256
_NC = 2
_NS = 16
_NW = _NC * _NS
_EP = 327680
_RPT = _EP // _NW // 128
_ACC_N = _N + 16
_NBLK = _ACC_N // 16
_OBLK = _N // 16


def _zero_acc(zbuf, acc_ref, s, nblk):
    def _zrow(i, _):
        zbuf[i // 8, pl.ds((i % 8) * 16, 16)] = jnp.zeros((16,), jnp.float32)
        return 0
    lax.fori_loop(0, 128, _zrow, 0)

    def _zcp(j, _):
        b = s + j * _NS
        @pl.when(b < nblk)
        def _():
            pltpu.sync_copy(zbuf.at[pl.ds(0, 16)],
                            acc_ref.at[pl.ds(pl.multiple_of(b * 16, 16), 16)])
        return 0
    lax.fori_loop(0, (nblk + _NS - 1) // _NS, _zcp, 0)


def _mul_rows(arow, brow):
    def _mul(r, _):
        for k in range(8):
            sl = pl.ds(k * 16, 16)
            arow[r, sl] = arow[r, sl] * brow[r, sl]
        return 0
    lax.fori_loop(0, 128, _mul, 0)


def _msg_sc_body(h_hbm, e_hbm, src_hbm, dst_hbm, out_hbm,
                 srcv, dstv, hrow, erow, acc_ref, sem):
    c = lax.axis_index("c")
    s = lax.axis_index("s")
    wid = c * _NS + s

    _zero_acc(erow, acc_ref, s, _NBLK)
    plsc.subcore_barrier()

    def _edge_chunk(lr, _):
        pltpu.sync_copy(src_hbm.at[wid, lr], srcv)
        pltpu.sync_copy(dst_hbm.at[wid, lr], dstv)
        pltpu.async_copy(h_hbm.at[srcv.at[0]], hrow, sem).wait()
        pltpu.sync_copy(e_hbm.at[wid * _RPT + lr], erow)
        _mul_rows(hrow, erow)
        pltpu.sync_copy(hrow, acc_ref.at[dstv.at[0]], add=True)
        return 0
    lax.fori_loop(0, _RPT, _edge_chunk, 0)
    plsc.subcore_barrier()

    def _ocp(j, _):
        b = s + j * _NS
        @pl.when(b < _OBLK)
        def _():
            pltpu.sync_copy(acc_ref.at[pl.ds(pl.multiple_of(b * 16, 16), 16)],
                            out_hbm.at[c, b])
        return 0
    lax.fori_loop(0, (_OBLK + _NS - 1) // _NS, _ocp, 0)


def _msg_sc(h, e3, src4, dst4):
    mesh = plsc.VectorSubcoreMesh(core_axis_name="c", subcore_axis_name="s")
    f = pl.kernel(
        _msg_sc_body,
        out_type=jax.ShapeDtypeStruct((_NC, _OBLK, 16, _D), jnp.float32),
        mesh=mesh,
        scratch_types=[
            pltpu.VMEM((1, 128), jnp.int32),
            pltpu.VMEM((1, 128), jnp.int32),
            pltpu.VMEM((128, _D), jnp.float32),
            pltpu.VMEM((128, _D), jnp.float32),
            pltpu.VMEM_SHARED((_ACC_N, _D), jnp.float32),
            pltpu.SemaphoreType.DMA,
        ],
    )
    return f(h, e3, src4, dst4)


def _qk_sc_body(q_hbm, k_hbm, src_hbm, dst_hbm, out_hbm,
                srcv, dstv, qrow, krow, sem):
    c = lax.axis_index("c")
    s = lax.axis_index("s")
    wid = c * _NS + s

    def _edge_chunk(lr, _):
        pltpu.sync_copy(src_hbm.at[wid, lr], srcv)
        pltpu.sync_copy(dst_hbm.at[wid, lr], dstv)
        cp1 = pltpu.async_copy(q_hbm.at[dstv.at[0]], qrow, sem)
        cp2 = pltpu.async_copy(k_hbm.at[srcv.at[0]], krow, sem)
        cp1.wait()
        cp2.wait()
        _mul_rows(qrow, krow)
        pltpu.sync_copy(qrow, out_hbm.at[wid * _RPT + lr])
        return 0
    lax.fori_loop(0, _RPT, _edge_chunk, 0)


def _qk_sc(qp, kp, src4, dst4):
    mesh = plsc.VectorSubcoreMesh(core_axis_name="c", subcore_axis_name="s")
    f = pl.kernel(
        _qk_sc_body,
        out_type=jax.ShapeDtypeStruct((_EP // 128, 128, _D), jnp.float32),
        mesh=mesh,
        scratch_types=[
            pltpu.VMEM((1, 128), jnp.int32),
            pltpu.VMEM((1, 128), jnp.int32),
            pltpu.VMEM((128, _D), jnp.float32),
            pltpu.VMEM((128, _D), jnp.float32),
            pltpu.SemaphoreType.DMA,
        ],
    )
    return f(qp, kp, src4, dst4)


def _den_sc_body(ex_hbm, dst_hbm, out_hbm, dstv, xrow, acc_ref, sem):
    c = lax.axis_index("c")
    s = lax.axis_index("s")
    wid = c * _NS + s

    _zero_acc(xrow, acc_ref, s, _NBLK)
    plsc.subcore_barrier()

    def _edge_chunk(lr, _):
        pltpu.sync_copy(dst_hbm.at[wid, lr], dstv)
        pltpu.sync_copy(ex_hbm.at[wid * _RPT + lr], xrow)
        pltpu.sync_copy(xrow, acc_ref.at[dstv.at[0]], add=True)
        return 0
    lax.fori_loop(0, _RPT, _edge_chunk, 0)
    plsc.subcore_barrier()

    def _ocp(j, _):
        b = s + j * _NS
        @pl.when(b < _OBLK)
        def _():
            pltpu.sync_copy(acc_ref.at[pl.ds(pl.multiple_of(b * 16, 16), 16)],
                            out_hbm.at[c, b])
        return 0
    lax.fori_loop(0, (_OBLK + _NS - 1) // _NS, _ocp, 0)


def _den_sc(ex3, dst4):
    mesh = plsc.VectorSubcoreMesh(core_axis_name="c", subcore_axis_name="s")
    f = pl.kernel(
        _den_sc_body,
        out_type=jax.ShapeDtypeStruct((_NC, _OBLK, 16, _D), jnp.float32),
        mesh=mesh,
        scratch_types=[
            pltpu.VMEM((1, 128), jnp.int32),
            pltpu.VMEM((128, _D), jnp.float32),
            pltpu.VMEM_SHARED((_ACC_N, _D), jnp.float32),
            pltpu.SemaphoreType.DMA,
        ],
    )
    return f(ex3, dst4)


_HI = jax.lax.Precision.HIGHEST


def _dot(a, b):
    return jax.lax.dot_general(a, b, (((1,), (0,)), ((), ())))


def _dotx(a, b):
    return jax.lax.dot_general(a, b, (((1,), (0,)), ((), ())), precision=_HI)


def _pre_tc_body(pe_ref, wp_ref, bp_ref, wr_ref, br_ref, h_ref):
    t = _dot(pe_ref[...], wp_ref[...])
    h1 = jnp.maximum(t + bp_ref[...], 0.0) + jnp.maximum(-t + bp_ref[...], 0.0)
    h_ref[...] = _dot(h1, wr_ref[...]) + br_ref[...]


def _pre_tc(pe, W_phi, b_phi, W_rho, b_rho):
    P = pe.shape[1]
    return pl.pallas_call(
        _pre_tc_body,
        grid=(10,),
        in_specs=[
            pl.BlockSpec((1000, P), lambda i: (i, 0)),
            pl.BlockSpec((P, _D), lambda i: (0, 0)),
            pl.BlockSpec((1, _D), lambda i: (0, 0)),
            pl.BlockSpec((_D, _D), lambda i: (0, 0)),
            pl.BlockSpec((1, _D), lambda i: (0, 0)),
        ],
        out_specs=pl.BlockSpec((1000, _D), lambda i: (i, 0)),
        out_shape=jax.ShapeDtypeStruct((_N, _D), jnp.float32),
    )(pe, W_phi, b_phi.reshape(1, _D), W_rho, b_rho.reshape(1, _D))


def _edge_tc_body(ea_ref, we_ref, be_ref, e_ref):
    e_ref[...] = jnp.maximum(_dot(ea_ref[...], we_ref[...]) + be_ref[...], 0.0)


def _edge_tc(ea_pad, W_edge, b_edge):
    ED = ea_pad.shape[1]
    return pl.pallas_call(
        _edge_tc_body,
        grid=(_EP // 2048,),
        in_specs=[
            pl.BlockSpec((2048, ED), lambda i: (i, 0)),
            pl.BlockSpec((ED, _D), lambda i: (0, 0)),
            pl.BlockSpec((1, _D), lambda i: (0, 0)),
        ],
        out_specs=pl.BlockSpec((2048, _D), lambda i: (i, 0)),
        out_shape=jax.ShapeDtypeStruct((_EP, _D), jnp.float32),
    )(ea_pad, W_edge, b_edge.reshape(1, _D))


def _qkv_tc_body(x_ref, h_ref, m0_ref, m1_ref, wq_ref, bq_ref, wk_ref, bk_ref,
                 wv_ref, bv_ref, q_ref, k_ref, v_ref):
    hn = x_ref[...] + h_ref[...] + m0_ref[...] + m1_ref[...]
    q_ref[...] = _dot(hn, wq_ref[...]) + bq_ref[...]
    k_ref[...] = _dot(hn, wk_ref[...]) + bk_ref[...]
    v_ref[...] = _dot(hn, wv_ref[...]) + bv_ref[...]


def _qkv_tc(x, h, m0, m1, Wq, bq, Wk, bk, Wv, bv):
    blk = pl.BlockSpec((1000, _D), lambda i: (i, 0))
    wblk = pl.BlockSpec((_D, _D), lambda i: (0, 0))
    bblk = pl.BlockSpec((1, _D), lambda i: (0, 0))
    sds = jax.ShapeDtypeStruct((_N, _D), jnp.float32)
    return pl.pallas_call(
        _qkv_tc_body,
        grid=(10,),
        in_specs=[blk, blk, blk, blk, wblk, bblk, wblk, bblk, wblk, bblk],
        out_specs=(blk, blk, blk),
        out_shape=(sds, sds, sds),
    )(x, h, m0, m1, Wq, bq.reshape(1, _D), Wk, bk.reshape(1, _D),
      Wv, bv.reshape(1, _D))


def _ex_tc_body(prod_ref, ex_ref):
    hid = lax.broadcasted_iota(jnp.int32, (_D, _H), 0) // (_D // _H)
    red = (hid == lax.broadcasted_iota(jnp.int32, (_D, _H), 1)).astype(jnp.float32)
    score = _dotx(prod_ref[...], red) * 0.25
    exs = jnp.exp(score)
    expand = (lax.broadcasted_iota(jnp.int32, (_H, _D), 0)
              == lax.broadcasted_iota(jnp.int32, (_H, _D), 1) // (_D // _H)
              ).astype(jnp.float32)
    ex_ref[...] = _dotx(exs, expand)


def _ex_tc(prod):
    return pl.pallas_call(
        _ex_tc_body,
        grid=(_EP // 2048,),
        in_specs=[pl.BlockSpec((2048, _D), lambda i: (i, 0))],
        out_specs=pl.BlockSpec((2048, _D), lambda i: (i, 0)),
        out_shape=jax.ShapeDtypeStruct((_EP, _D), jnp.float32),
    )(prod)


def _fin_tc_body(attn_ref, denw_ref, batch_ref, wr1_ref, br1_ref, wr2_ref,
                 br2_ref, out_ref, accs_ref, accc_ref):
    i = pl.program_id(0)
    node = attn_ref[...] / (denw_ref[...] + 1e-16)
    b = batch_ref[0, 0, :]
    onehot_t = (lax.broadcasted_iota(jnp.int32, (_G, 1000), 0)
                == b[None, :]).astype(jnp.float32)
    ns = _dotx(onehot_t, node)
    nc = _dotx(onehot_t, jnp.ones((1000, _D), jnp.float32))

    @pl.when(i == 0)
    def _():
        accs_ref[...] = ns
        accc_ref[...] = nc

    @pl.when(i > 0)
    def _():
        accs_ref[...] = accs_ref[...] + ns
        accc_ref[...] = accc_ref[...] + nc

    @pl.when(i == 9)
    def _():
        graph = accs_ref[...] / jnp.maximum(accc_ref[...], 1.0)
        r = jnp.maximum(_dot(graph, wr1_ref[...]) + br1_ref[...], 0.0)
        out_ref[...] = _dot(r, wr2_ref[...]) + br2_ref[...]


def _fin_tc(attn, denw, batch, Wr1, br1, Wr2, br2):
    D2 = _D // 2
    return pl.pallas_call(
        _fin_tc_body,
        grid=(10,),
        in_specs=[
            pl.BlockSpec((1000, _D), lambda i: (i, 0)),
            pl.BlockSpec((1000, _D), lambda i: (i, 0)),
            pl.BlockSpec((1, 1, 1000), lambda i: (i, 0, 0)),
            pl.BlockSpec((_D, D2), lambda i: (0, 0)),
            pl.BlockSpec((1, D2), lambda i: (0, 0)),
            pl.BlockSpec((D2, 1), lambda i: (0, 0)),
            pl.BlockSpec((1, 1), lambda i: (0, 0)),
        ],
        out_specs=pl.BlockSpec((_G, 1), lambda i: (0, 0)),
        out_shape=jax.ShapeDtypeStruct((_G, 1), jnp.float32),
        scratch_shapes=[
            pltpu.VMEM((_G, _D), jnp.float32),
            pltpu.VMEM((_G, _D), jnp.float32),
        ],
    )(attn, denw, batch.reshape(10, 1, 1000), Wr1, br1.reshape(1, D2),
      Wr2, br2.reshape(1, 1))


def kernel(x, edge_index, pe, edge_attr, batch, W_phi, b_phi, W_rho, b_rho, W_edge, b_edge, Wq, bq, Wk, bk, Wv, bv, Wr1, br1, Wr2, br2):
    N, D = x.shape
    H = _H
    DH = D // H
    G = _G
    src = edge_index[0]
    dst = edge_index[1]
    npad = _EP - _E
    src4 = jnp.concatenate([src, jnp.zeros((npad,), jnp.int32)]).reshape(_NW, _RPT, 1, 128)
    dst4 = jnp.concatenate([dst, jnp.full((npad,), _N, jnp.int32)]).reshape(_NW, _RPT, 1, 128)

    h = _pre_tc(pe, W_phi, b_phi, W_rho, b_rho)
    ea_pad = jnp.concatenate([edge_attr, jnp.zeros((npad, edge_attr.shape[1]), jnp.float32)])
    e = _edge_tc(ea_pad, W_edge, b_edge)
    e3 = e.reshape(_EP // 128, 128, _D)

    m2 = _msg_sc(h, e3, src4, dst4)

    q, k, v = _qkv_tc(x, h, m2[0].reshape(N, D), m2[1].reshape(N, D),
                      Wq, bq, Wk, bk, Wv, bv)
    zpad = jnp.zeros((_ACC_N - N, D), jnp.float32)
    qp = jnp.concatenate([q, zpad])
    kp = jnp.concatenate([k, zpad])
    vp = jnp.concatenate([v, zpad])

    prod3 = _qk_sc(qp, kp, src4, dst4)

    exwide = _ex_tc(prod3.reshape(_EP, _D))
    ex3 = exwide.reshape(_EP // 128, 128, _D)

    attn2 = _msg_sc(vp, ex3, src4, dst4)
    attn = (attn2[0] + attn2[1]).reshape(N, D)
    den2 = _den_sc(ex3, dst4)
    denw = (den2[0] + den2[1]).reshape(N, D)

    return _fin_tc(attn, denw, batch, Wr1, br1, Wr2, br2)


# speedup vs baseline: 1.0453x; 1.0453x over previous
import jax
import jax.numpy as jnp
from jax import lax
from jax.experimental import pallas as pl
from jax.experimental.pallas import tpu as pltpu
from jax.experimental.pallas import tpu_sc as plsc

_N = 10000
_E = 320000
_D = 128
_H = 8
_G = 256
_NC = 2
_NS = 16
_NW = _NC * _NS
_EP = 327680
_RPT = _EP // _NW // 128
_ACC_N = _N + 16
_NBLK = _ACC_N // 16
_OBLK = _N // 16


def _zero_acc(zbuf, acc_ref, s, nblk):
    def _zrow(i, _):
        zbuf[i // 8, pl.ds((i % 8) * 16, 16)] = jnp.zeros((16,), jnp.float32)
        return 0
    lax.fori_loop(0, 128, _zrow, 0)

    def _zcp(j, _):
        b = s + j * _NS
        @pl.when(b < nblk)
        def _():
            pltpu.sync_copy(zbuf.at[pl.ds(0, 16)],
                            acc_ref.at[pl.ds(pl.multiple_of(b * 16, 16), 16)])
        return 0
    lax.fori_loop(0, (nblk + _NS - 1) // _NS, _zcp, 0)


def _mul_rows(arow, brow):
    def _mul(r, _):
        for k in range(8):
            sl = pl.ds(k * 16, 16)
            arow[r, sl] = arow[r, sl] * brow[r, sl]
        return 0
    lax.fori_loop(0, 128, _mul, 0)


def _msg_sc_body(h_hbm, e_hbm, src_hbm, dst_hbm, out_hbm,
                 srcv, dstv, hrow, erow, acc_ref, sem, sem2):
    c = lax.axis_index("c")
    s = lax.axis_index("s")
    wid = c * _NS + s

    _zero_acc(erow, acc_ref, s, _NBLK)
    plsc.subcore_barrier()

    def _edge_chunk(lr, _):
        cpe = pltpu.async_copy(e_hbm.at[wid * _RPT + lr], erow, sem2)
        pltpu.sync_copy(src_hbm.at[wid, lr], srcv)
        pltpu.sync_copy(dst_hbm.at[wid, lr], dstv)
        cph = pltpu.async_copy(h_hbm.at[srcv.at[0]], hrow, sem)
        cph.wait()
        cpe.wait()
        _mul_rows(hrow, erow)
        pltpu.sync_copy(hrow, acc_ref.at[dstv.at[0]], add=True)
        return 0
    lax.fori_loop(0, _RPT, _edge_chunk, 0)
    plsc.subcore_barrier()

    def _ocp(j, _):
        b = s + j * _NS
        @pl.when(b < _OBLK)
        def _():
            pltpu.sync_copy(acc_ref.at[pl.ds(pl.multiple_of(b * 16, 16), 16)],
                            out_hbm.at[c, b])
        return 0
    lax.fori_loop(0, (_OBLK + _NS - 1) // _NS, _ocp, 0)


def _msg_sc(h, e3, src4, dst4):
    mesh = plsc.VectorSubcoreMesh(core_axis_name="c", subcore_axis_name="s")
    f = pl.kernel(
        _msg_sc_body,
        out_type=jax.ShapeDtypeStruct((_NC, _OBLK, 16, _D), jnp.float32),
        mesh=mesh,
        scratch_types=[
            pltpu.VMEM((1, 128), jnp.int32),
            pltpu.VMEM((1, 128), jnp.int32),
            pltpu.VMEM((128, _D), jnp.float32),
            pltpu.VMEM((128, _D), jnp.float32),
            pltpu.VMEM_SHARED((_ACC_N, _D), jnp.float32),
            pltpu.SemaphoreType.DMA,
            pltpu.SemaphoreType.DMA,
        ],
    )
    return f(h, e3, src4, dst4)


def _qk_sc_body(q_hbm, k_hbm, src_hbm, dst_hbm, out_hbm,
                srcv, dstv, qrow, krow, sem):
    c = lax.axis_index("c")
    s = lax.axis_index("s")
    wid = c * _NS + s

    def _edge_chunk(lr, _):
        pltpu.sync_copy(src_hbm.at[wid, lr], srcv)
        pltpu.sync_copy(dst_hbm.at[wid, lr], dstv)
        cp1 = pltpu.async_copy(q_hbm.at[dstv.at[0]], qrow, sem)
        cp2 = pltpu.async_copy(k_hbm.at[srcv.at[0]], krow, sem)
        cp1.wait()
        cp2.wait()
        _mul_rows(qrow, krow)
        pltpu.sync_copy(qrow, out_hbm.at[wid * _RPT + lr])
        return 0
    lax.fori_loop(0, _RPT, _edge_chunk, 0)


def _qk_sc(qp, kp, src4, dst4):
    mesh = plsc.VectorSubcoreMesh(core_axis_name="c", subcore_axis_name="s")
    f = pl.kernel(
        _qk_sc_body,
        out_type=jax.ShapeDtypeStruct((_EP // 128, 128, _D), jnp.float32),
        mesh=mesh,
        scratch_types=[
            pltpu.VMEM((1, 128), jnp.int32),
            pltpu.VMEM((1, 128), jnp.int32),
            pltpu.VMEM((128, _D), jnp.float32),
            pltpu.VMEM((128, _D), jnp.float32),
            pltpu.SemaphoreType.DMA,
        ],
    )
    return f(qp, kp, src4, dst4)


def _den_sc_body(ex_hbm, dst_hbm, out_hbm, dstv, xrow, acc_ref, sem):
    c = lax.axis_index("c")
    s = lax.axis_index("s")
    wid = c * _NS + s

    _zero_acc(xrow, acc_ref, s, _NBLK)
    plsc.subcore_barrier()

    def _edge_chunk(lr, _):
        pltpu.sync_copy(dst_hbm.at[wid, lr], dstv)
        pltpu.sync_copy(ex_hbm.at[wid * _RPT + lr], xrow)
        pltpu.sync_copy(xrow, acc_ref.at[dstv.at[0]], add=True)
        return 0
    lax.fori_loop(0, _RPT, _edge_chunk, 0)
    plsc.subcore_barrier()

    def _ocp(j, _):
        b = s + j * _NS
        @pl.when(b < _OBLK)
        def _():
            pltpu.sync_copy(acc_ref.at[pl.ds(pl.multiple_of(b * 16, 16), 16)],
                            out_hbm.at[c, b])
        return 0
    lax.fori_loop(0, (_OBLK + _NS - 1) // _NS, _ocp, 0)


def _den_sc(ex3, dst4):
    mesh = plsc.VectorSubcoreMesh(core_axis_name="c", subcore_axis_name="s")
    f = pl.kernel(
        _den_sc_body,
        out_type=jax.ShapeDtypeStruct((_NC, _OBLK, 16, _D), jnp.float32),
        mesh=mesh,
        scratch_types=[
            pltpu.VMEM((1, 128), jnp.int32),
            pltpu.VMEM((128, _D), jnp.float32),
            pltpu.VMEM_SHARED((_ACC_N, _D), jnp.float32),
            pltpu.SemaphoreType.DMA,
        ],
    )
    return f(ex3, dst4)


_HI = jax.lax.Precision.HIGHEST


def _dot(a, b):
    return jax.lax.dot_general(a, b, (((1,), (0,)), ((), ())))


def _dotx(a, b):
    return jax.lax.dot_general(a, b, (((1,), (0,)), ((), ())), precision=_HI)


def _pre_tc_body(pe_ref, wp_ref, bp_ref, wr_ref, br_ref, h_ref):
    t = _dot(pe_ref[...], wp_ref[...])
    h1 = jnp.maximum(t + bp_ref[...], 0.0) + jnp.maximum(-t + bp_ref[...], 0.0)
    h_ref[...] = _dot(h1, wr_ref[...]) + br_ref[...]


def _pre_tc(pe, W_phi, b_phi, W_rho, b_rho):
    P = pe.shape[1]
    return pl.pallas_call(
        _pre_tc_body,
        grid=(10,),
        in_specs=[
            pl.BlockSpec((1000, P), lambda i: (i, 0)),
            pl.BlockSpec((P, _D), lambda i: (0, 0)),
            pl.BlockSpec((1, _D), lambda i: (0, 0)),
            pl.BlockSpec((_D, _D), lambda i: (0, 0)),
            pl.BlockSpec((1, _D), lambda i: (0, 0)),
        ],
        out_specs=pl.BlockSpec((1000, _D), lambda i: (i, 0)),
        out_shape=jax.ShapeDtypeStruct((_N, _D), jnp.float32),
    )(pe, W_phi, b_phi.reshape(1, _D), W_rho, b_rho.reshape(1, _D))


def _edge_tc_body(ea_ref, we_ref, be_ref, e_ref):
    e_ref[...] = jnp.maximum(_dot(ea_ref[...], we_ref[...]) + be_ref[...], 0.0)


def _edge_tc(ea_pad, W_edge, b_edge):
    ED = ea_pad.shape[1]
    return pl.pallas_call(
        _edge_tc_body,
        grid=(_EP // 2048,),
        in_specs=[
            pl.BlockSpec((2048, ED), lambda i: (i, 0)),
            pl.BlockSpec((ED, _D), lambda i: (0, 0)),
            pl.BlockSpec((1, _D), lambda i: (0, 0)),
        ],
        out_specs=pl.BlockSpec((2048, _D), lambda i: (i, 0)),
        out_shape=jax.ShapeDtypeStruct((_EP, _D), jnp.float32),
    )(ea_pad, W_edge, b_edge.reshape(1, _D))


def _qkv_tc_body(x_ref, h_ref, m0_ref, m1_ref, wq_ref, bq_ref, wk_ref, bk_ref,
                 wv_ref, bv_ref, q_ref, k_ref, v_ref):
    hn = x_ref[...] + h_ref[...] + m0_ref[...] + m1_ref[...]
    q_ref[...] = _dot(hn, wq_ref[...]) + bq_ref[...]
    k_ref[...] = _dot(hn, wk_ref[...]) + bk_ref[...]
    v_ref[...] = _dot(hn, wv_ref[...]) + bv_ref[...]


def _qkv_tc(x, h, m0, m1, Wq, bq, Wk, bk, Wv, bv):
    blk = pl.BlockSpec((1000, _D), lambda i: (i, 0))
    wblk = pl.BlockSpec((_D, _D), lambda i: (0, 0))
    bblk = pl.BlockSpec((1, _D), lambda i: (0, 0))
    sds = jax.ShapeDtypeStruct((_N, _D), jnp.float32)
    return pl.pallas_call(
        _qkv_tc_body,
        grid=(10,),
        in_specs=[blk, blk, blk, blk, wblk, bblk, wblk, bblk, wblk, bblk],
        out_specs=(blk, blk, blk),
        out_shape=(sds, sds, sds),
    )(x, h, m0, m1, Wq, bq.reshape(1, _D), Wk, bk.reshape(1, _D),
      Wv, bv.reshape(1, _D))


def _ex_tc_body(prod_ref, ex_ref):
    hid = lax.broadcasted_iota(jnp.int32, (_D, _H), 0) // (_D // _H)
    red = (hid == lax.broadcasted_iota(jnp.int32, (_D, _H), 1)).astype(jnp.float32)
    score = _dotx(prod_ref[...], red) * 0.25
    exs = jnp.exp(score)
    expand = (lax.broadcasted_iota(jnp.int32, (_H, _D), 0)
              == lax.broadcasted_iota(jnp.int32, (_H, _D), 1) // (_D // _H)
              ).astype(jnp.float32)
    ex_ref[...] = _dotx(exs, expand)


def _ex_tc(prod):
    return pl.pallas_call(
        _ex_tc_body,
        grid=(_EP // 2048,),
        in_specs=[pl.BlockSpec((2048, _D), lambda i: (i, 0))],
        out_specs=pl.BlockSpec((2048, _D), lambda i: (i, 0)),
        out_shape=jax.ShapeDtypeStruct((_EP, _D), jnp.float32),
    )(prod)


def _fin_tc_body(attn_ref, denw_ref, batch_ref, wr1_ref, br1_ref, wr2_ref,
                 br2_ref, out_ref, accs_ref, accc_ref):
    i = pl.program_id(0)
    node = attn_ref[...] / (denw_ref[...] + 1e-16)
    b = batch_ref[0, 0, :]
    onehot_t = (lax.broadcasted_iota(jnp.int32, (_G, 1000), 0)
                == b[None, :]).astype(jnp.float32)
    ns = _dotx(onehot_t, node)
    nc = _dotx(onehot_t, jnp.ones((1000, _D), jnp.float32))

    @pl.when(i == 0)
    def _():
        accs_ref[...] = ns
        accc_ref[...] = nc

    @pl.when(i > 0)
    def _():
        accs_ref[...] = accs_ref[...] + ns
        accc_ref[...] = accc_ref[...] + nc

    @pl.when(i == 9)
    def _():
        graph = accs_ref[...] / jnp.maximum(accc_ref[...], 1.0)
        r = jnp.maximum(_dot(graph, wr1_ref[...]) + br1_ref[...], 0.0)
        out_ref[...] = _dot(r, wr2_ref[...]) + br2_ref[...]


def _fin_tc(attn, denw, batch, Wr1, br1, Wr2, br2):
    D2 = _D // 2
    return pl.pallas_call(
        _fin_tc_body,
        grid=(10,),
        in_specs=[
            pl.BlockSpec((1000, _D), lambda i: (i, 0)),
            pl.BlockSpec((1000, _D), lambda i: (i, 0)),
            pl.BlockSpec((1, 1, 1000), lambda i: (i, 0, 0)),
            pl.BlockSpec((_D, D2), lambda i: (0, 0)),
            pl.BlockSpec((1, D2), lambda i: (0, 0)),
            pl.BlockSpec((D2, 1), lambda i: (0, 0)),
            pl.BlockSpec((1, 1), lambda i: (0, 0)),
        ],
        out_specs=pl.BlockSpec((_G, 1), lambda i: (0, 0)),
        out_shape=jax.ShapeDtypeStruct((_G, 1), jnp.float32),
        scratch_shapes=[
            pltpu.VMEM((_G, _D), jnp.float32),
            pltpu.VMEM((_G, _D), jnp.float32),
        ],
    )(attn, denw, batch.reshape(10, 1, 1000), Wr1, br1.reshape(1, D2),
      Wr2, br2.reshape(1, 1))


def kernel(x, edge_index, pe, edge_attr, batch, W_phi, b_phi, W_rho, b_rho, W_edge, b_edge, Wq, bq, Wk, bk, Wv, bv, Wr1, br1, Wr2, br2):
    N, D = x.shape
    H = _H
    DH = D // H
    G = _G
    src = edge_index[0]
    dst = edge_index[1]
    npad = _EP - _E
    src4 = jnp.concatenate([src, jnp.zeros((npad,), jnp.int32)]).reshape(_NW, _RPT, 1, 128)
    dst4 = jnp.concatenate([dst, jnp.full((npad,), _N, jnp.int32)]).reshape(_NW, _RPT, 1, 128)

    h = _pre_tc(pe, W_phi, b_phi, W_rho, b_rho)
    ea_pad = jnp.concatenate([edge_attr, jnp.zeros((npad, edge_attr.shape[1]), jnp.float32)])
    e = _edge_tc(ea_pad, W_edge, b_edge)
    e3 = e.reshape(_EP // 128, 128, _D)

    m2 = _msg_sc(h, e3, src4, dst4)

    q, k, v = _qkv_tc(x, h, m2[0].reshape(N, D), m2[1].reshape(N, D),
                      Wq, bq, Wk, bk, Wv, bv)
    zpad = jnp.zeros((_ACC_N - N, D), jnp.float32)
    qp = jnp.concatenate([q, zpad])
    kp = jnp.concatenate([k, zpad])
    vp = jnp.concatenate([v, zpad])

    prod3 = _qk_sc(qp, kp, src4, dst4)

    exwide = _ex_tc(prod3.reshape(_EP, _D))
    ex3 = exwide.reshape(_EP // 128, 128, _D)

    attn2 = _msg_sc(vp, ex3, src4, dst4)
    attn = (attn2[0] + attn2[1]).reshape(N, D)
    den2 = _den_sc(ex3, dst4)
    denw = (den2[0] + den2[1]).reshape(N, D)

    return _fin_tc(attn, denw, batch, Wr1, br1, Wr2, br2)


# speedup vs baseline: 1.0465x; 1.0012x over previous
import jax
import jax.numpy as jnp
from jax import lax
from jax.experimental import pallas as pl
from jax.experimental.pallas import tpu as pltpu
from jax.experimental.pallas import tpu_sc as plsc

_N = 10000
_E = 320000
_D = 128
_H = 8
_G = 256
_NC = 2
_NS = 16
_NW = _NC * _NS
_EP = 327680
_RPT = _EP // _NW // 128
_ACC_N = _N + 16
_NBLK = _ACC_N // 16
_OBLK = _N // 16


def _zero_acc(zbuf, acc_ref, s, nblk):
    def _zrow(i, _):
        zbuf[i // 8, pl.ds((i % 8) * 16, 16)] = jnp.zeros((16,), jnp.float32)
        return 0
    lax.fori_loop(0, 128, _zrow, 0)

    def _zcp(j, _):
        b = s + j * _NS
        @pl.when(b < nblk)
        def _():
            pltpu.sync_copy(zbuf.at[pl.ds(0, 16)],
                            acc_ref.at[pl.ds(pl.multiple_of(b * 16, 16), 16)])
        return 0
    lax.fori_loop(0, (nblk + _NS - 1) // _NS, _zcp, 0)


def _mul_rows(arow, brow):
    def _mul(r4, _):
        for rr in range(4):
            r = r4 * 4 + rr
            for k in range(8):
                sl = pl.ds(k * 16, 16)
                arow[r, sl] = arow[r, sl] * brow[r, sl]
        return 0
    lax.fori_loop(0, 32, _mul, 0)


def _msg_sc_body(h_hbm, e_hbm, src_hbm, dst_hbm, out_hbm,
                 srcv, dstv, hrow, erow, acc_ref, sem, sem2):
    c = lax.axis_index("c")
    s = lax.axis_index("s")
    wid = c * _NS + s

    _zero_acc(erow, acc_ref, s, _NBLK)
    plsc.subcore_barrier()

    def _edge_chunk(lr, _):
        cpe = pltpu.async_copy(e_hbm.at[wid * _RPT + lr], erow, sem2)
        pltpu.sync_copy(src_hbm.at[wid, lr], srcv)
        pltpu.sync_copy(dst_hbm.at[wid, lr], dstv)
        cph = pltpu.async_copy(h_hbm.at[srcv.at[0]], hrow, sem)
        cph.wait()
        cpe.wait()
        _mul_rows(hrow, erow)
        pltpu.sync_copy(hrow, acc_ref.at[dstv.at[0]], add=True)
        return 0
    lax.fori_loop(0, _RPT, _edge_chunk, 0)
    plsc.subcore_barrier()

    def _ocp(j, _):
        b = s + j * _NS
        @pl.when(b < _OBLK)
        def _():
            pltpu.sync_copy(acc_ref.at[pl.ds(pl.multiple_of(b * 16, 16), 16)],
                            out_hbm.at[c, b])
        return 0
    lax.fori_loop(0, (_OBLK + _NS - 1) // _NS, _ocp, 0)


def _msg_sc(h, e3, src4, dst4):
    mesh = plsc.VectorSubcoreMesh(core_axis_name="c", subcore_axis_name="s")
    f = pl.kernel(
        _msg_sc_body,
        out_type=jax.ShapeDtypeStruct((_NC, _OBLK, 16, _D), jnp.float32),
        mesh=mesh,
        scratch_types=[
            pltpu.VMEM((1, 128), jnp.int32),
            pltpu.VMEM((1, 128), jnp.int32),
            pltpu.VMEM((128, _D), jnp.float32),
            pltpu.VMEM((128, _D), jnp.float32),
            pltpu.VMEM_SHARED((_ACC_N, _D), jnp.float32),
            pltpu.SemaphoreType.DMA,
            pltpu.SemaphoreType.DMA,
        ],
    )
    return f(h, e3, src4, dst4)


def _qk_sc_body(q_hbm, k_hbm, src_hbm, dst_hbm, out_hbm,
                srcv, dstv, qrow, krow, sem):
    c = lax.axis_index("c")
    s = lax.axis_index("s")
    wid = c * _NS + s

    def _edge_chunk(lr, _):
        pltpu.sync_copy(src_hbm.at[wid, lr], srcv)
        pltpu.sync_copy(dst_hbm.at[wid, lr], dstv)
        cp1 = pltpu.async_copy(q_hbm.at[dstv.at[0]], qrow, sem)
        cp2 = pltpu.async_copy(k_hbm.at[srcv.at[0]], krow, sem)
        cp1.wait()
        cp2.wait()
        _mul_rows(qrow, krow)
        pltpu.sync_copy(qrow, out_hbm.at[wid * _RPT + lr])
        return 0
    lax.fori_loop(0, _RPT, _edge_chunk, 0)


def _qk_sc(qp, kp, src4, dst4):
    mesh = plsc.VectorSubcoreMesh(core_axis_name="c", subcore_axis_name="s")
    f = pl.kernel(
        _qk_sc_body,
        out_type=jax.ShapeDtypeStruct((_EP // 128, 128, _D), jnp.float32),
        mesh=mesh,
        scratch_types=[
            pltpu.VMEM((1, 128), jnp.int32),
            pltpu.VMEM((1, 128), jnp.int32),
            pltpu.VMEM((128, _D), jnp.float32),
            pltpu.VMEM((128, _D), jnp.float32),
            pltpu.SemaphoreType.DMA,
        ],
    )
    return f(qp, kp, src4, dst4)


def _den_sc_body(ex_hbm, dst_hbm, out_hbm, dstv, xrow, acc_ref, sem):
    c = lax.axis_index("c")
    s = lax.axis_index("s")
    wid = c * _NS + s

    _zero_acc(xrow, acc_ref, s, _NBLK)
    plsc.subcore_barrier()

    def _edge_chunk(lr, _):
        pltpu.sync_copy(dst_hbm.at[wid, lr], dstv)
        pltpu.sync_copy(ex_hbm.at[wid * _RPT + lr], xrow)
        pltpu.sync_copy(xrow, acc_ref.at[dstv.at[0]], add=True)
        return 0
    lax.fori_loop(0, _RPT, _edge_chunk, 0)
    plsc.subcore_barrier()

    def _ocp(j, _):
        b = s + j * _NS
        @pl.when(b < _OBLK)
        def _():
            pltpu.sync_copy(acc_ref.at[pl.ds(pl.multiple_of(b * 16, 16), 16)],
                            out_hbm.at[c, b])
        return 0
    lax.fori_loop(0, (_OBLK + _NS - 1) // _NS, _ocp, 0)


def _den_sc(ex3, dst4):
    mesh = plsc.VectorSubcoreMesh(core_axis_name="c", subcore_axis_name="s")
    f = pl.kernel(
        _den_sc_body,
        out_type=jax.ShapeDtypeStruct((_NC, _OBLK, 16, _D), jnp.float32),
        mesh=mesh,
        scratch_types=[
            pltpu.VMEM((1, 128), jnp.int32),
            pltpu.VMEM((128, _D), jnp.float32),
            pltpu.VMEM_SHARED((_ACC_N, _D), jnp.float32),
            pltpu.SemaphoreType.DMA,
        ],
    )
    return f(ex3, dst4)


_HI = jax.lax.Precision.HIGHEST


def _dot(a, b):
    return jax.lax.dot_general(a, b, (((1,), (0,)), ((), ())))


def _dotx(a, b):
    return jax.lax.dot_general(a, b, (((1,), (0,)), ((), ())), precision=_HI)


def _pre_tc_body(pe_ref, wp_ref, bp_ref, wr_ref, br_ref, h_ref):
    t = _dot(pe_ref[...], wp_ref[...])
    h1 = jnp.maximum(t + bp_ref[...], 0.0) + jnp.maximum(-t + bp_ref[...], 0.0)
    h_ref[...] = _dot(h1, wr_ref[...]) + br_ref[...]


def _pre_tc(pe, W_phi, b_phi, W_rho, b_rho):
    P = pe.shape[1]
    return pl.pallas_call(
        _pre_tc_body,
        grid=(10,),
        in_specs=[
            pl.BlockSpec((1000, P), lambda i: (i, 0)),
            pl.BlockSpec((P, _D), lambda i: (0, 0)),
            pl.BlockSpec((1, _D), lambda i: (0, 0)),
            pl.BlockSpec((_D, _D), lambda i: (0, 0)),
            pl.BlockSpec((1, _D), lambda i: (0, 0)),
        ],
        out_specs=pl.BlockSpec((1000, _D), lambda i: (i, 0)),
        out_shape=jax.ShapeDtypeStruct((_N, _D), jnp.float32),
    )(pe, W_phi, b_phi.reshape(1, _D), W_rho, b_rho.reshape(1, _D))


def _edge_tc_body(ea_ref, we_ref, be_ref, e_ref):
    e_ref[...] = jnp.maximum(_dot(ea_ref[...], we_ref[...]) + be_ref[...], 0.0)


def _edge_tc(ea_pad, W_edge, b_edge):
    ED = ea_pad.shape[1]
    return pl.pallas_call(
        _edge_tc_body,
        grid=(_EP // 2048,),
        in_specs=[
            pl.BlockSpec((2048, ED), lambda i: (i, 0)),
            pl.BlockSpec((ED, _D), lambda i: (0, 0)),
            pl.BlockSpec((1, _D), lambda i: (0, 0)),
        ],
        out_specs=pl.BlockSpec((2048, _D), lambda i: (i, 0)),
        out_shape=jax.ShapeDtypeStruct((_EP, _D), jnp.float32),
    )(ea_pad, W_edge, b_edge.reshape(1, _D))


def _qkv_tc_body(x_ref, h_ref, m0_ref, m1_ref, wq_ref, bq_ref, wk_ref, bk_ref,
                 wv_ref, bv_ref, q_ref, k_ref, v_ref):
    hn = x_ref[...] + h_ref[...] + m0_ref[...] + m1_ref[...]
    q_ref[...] = _dot(hn, wq_ref[...]) + bq_ref[...]
    k_ref[...] = _dot(hn, wk_ref[...]) + bk_ref[...]
    v_ref[...] = _dot(hn, wv_ref[...]) + bv_ref[...]


def _qkv_tc(x, h, m0, m1, Wq, bq, Wk, bk, Wv, bv):
    blk = pl.BlockSpec((1000, _D), lambda i: (i, 0))
    wblk = pl.BlockSpec((_D, _D), lambda i: (0, 0))
    bblk = pl.BlockSpec((1, _D), lambda i: (0, 0))
    sds = jax.ShapeDtypeStruct((_N, _D), jnp.float32)
    return pl.pallas_call(
        _qkv_tc_body,
        grid=(10,),
        in_specs=[blk, blk, blk, blk, wblk, bblk, wblk, bblk, wblk, bblk],
        out_specs=(blk, blk, blk),
        out_shape=(sds, sds, sds),
    )(x, h, m0, m1, Wq, bq.reshape(1, _D), Wk, bk.reshape(1, _D),
      Wv, bv.reshape(1, _D))


def _ex_tc_body(prod_ref, ex_ref):
    hid = lax.broadcasted_iota(jnp.int32, (_D, _H), 0) // (_D // _H)
    red = (hid == lax.broadcasted_iota(jnp.int32, (_D, _H), 1)).astype(jnp.float32)
    score = _dotx(prod_ref[...], red) * 0.25
    exs = jnp.exp(score)
    expand = (lax.broadcasted_iota(jnp.int32, (_H, _D), 0)
              == lax.broadcasted_iota(jnp.int32, (_H, _D), 1) // (_D // _H)
              ).astype(jnp.float32)
    ex_ref[...] = _dotx(exs, expand)


def _ex_tc(prod):
    return pl.pallas_call(
        _ex_tc_body,
        grid=(_EP // 2048,),
        in_specs=[pl.BlockSpec((2048, _D), lambda i: (i, 0))],
        out_specs=pl.BlockSpec((2048, _D), lambda i: (i, 0)),
        out_shape=jax.ShapeDtypeStruct((_EP, _D), jnp.float32),
    )(prod)


def _fin_tc_body(attn_ref, denw_ref, batch_ref, wr1_ref, br1_ref, wr2_ref,
                 br2_ref, out_ref, accs_ref, accc_ref):
    i = pl.program_id(0)
    node = attn_ref[...] / (denw_ref[...] + 1e-16)
    b = batch_ref[0, 0, :]
    onehot_t = (lax.broadcasted_iota(jnp.int32, (_G, 1000), 0)
                == b[None, :]).astype(jnp.float32)
    ns = _dotx(onehot_t, node)
    nc = _dotx(onehot_t, jnp.ones((1000, _D), jnp.float32))

    @pl.when(i == 0)
    def _():
        accs_ref[...] = ns
        accc_ref[...] = nc

    @pl.when(i > 0)
    def _():
        accs_ref[...] = accs_ref[...] + ns
        accc_ref[...] = accc_ref[...] + nc

    @pl.when(i == 9)
    def _():
        graph = accs_ref[...] / jnp.maximum(accc_ref[...], 1.0)
        r = jnp.maximum(_dot(graph, wr1_ref[...]) + br1_ref[...], 0.0)
        out_ref[...] = _dot(r, wr2_ref[...]) + br2_ref[...]


def _fin_tc(attn, denw, batch, Wr1, br1, Wr2, br2):
    D2 = _D // 2
    return pl.pallas_call(
        _fin_tc_body,
        grid=(10,),
        in_specs=[
            pl.BlockSpec((1000, _D), lambda i: (i, 0)),
            pl.BlockSpec((1000, _D), lambda i: (i, 0)),
            pl.BlockSpec((1, 1, 1000), lambda i: (i, 0, 0)),
            pl.BlockSpec((_D, D2), lambda i: (0, 0)),
            pl.BlockSpec((1, D2), lambda i: (0, 0)),
            pl.BlockSpec((D2, 1), lambda i: (0, 0)),
            pl.BlockSpec((1, 1), lambda i: (0, 0)),
        ],
        out_specs=pl.BlockSpec((_G, 1), lambda i: (0, 0)),
        out_shape=jax.ShapeDtypeStruct((_G, 1), jnp.float32),
        scratch_shapes=[
            pltpu.VMEM((_G, _D), jnp.float32),
            pltpu.VMEM((_G, _D), jnp.float32),
        ],
    )(attn, denw, batch.reshape(10, 1, 1000), Wr1, br1.reshape(1, D2),
      Wr2, br2.reshape(1, 1))


def kernel(x, edge_index, pe, edge_attr, batch, W_phi, b_phi, W_rho, b_rho, W_edge, b_edge, Wq, bq, Wk, bk, Wv, bv, Wr1, br1, Wr2, br2):
    N, D = x.shape
    H = _H
    DH = D // H
    G = _G
    src = edge_index[0]
    dst = edge_index[1]
    npad = _EP - _E
    src4 = jnp.concatenate([src, jnp.zeros((npad,), jnp.int32)]).reshape(_NW, _RPT, 1, 128)
    dst4 = jnp.concatenate([dst, jnp.full((npad,), _N, jnp.int32)]).reshape(_NW, _RPT, 1, 128)

    h = _pre_tc(pe, W_phi, b_phi, W_rho, b_rho)
    ea_pad = jnp.concatenate([edge_attr, jnp.zeros((npad, edge_attr.shape[1]), jnp.float32)])
    e = _edge_tc(ea_pad, W_edge, b_edge)
    e3 = e.reshape(_EP // 128, 128, _D)

    m2 = _msg_sc(h, e3, src4, dst4)

    q, k, v = _qkv_tc(x, h, m2[0].reshape(N, D), m2[1].reshape(N, D),
                      Wq, bq, Wk, bk, Wv, bv)
    zpad = jnp.zeros((_ACC_N - N, D), jnp.float32)
    qp = jnp.concatenate([q, zpad])
    kp = jnp.concatenate([k, zpad])
    vp = jnp.concatenate([v, zpad])

    prod3 = _qk_sc(qp, kp, src4, dst4)

    exwide = _ex_tc(prod3.reshape(_EP, _D))
    ex3 = exwide.reshape(_EP // 128, 128, _D)

    attn2 = _msg_sc(vp, ex3, src4, dst4)
    attn = (attn2[0] + attn2[1]).reshape(N, D)
    den2 = _den_sc(ex3, dst4)
    denw = (den2[0] + den2[1]).reshape(N, D)

    return _fin_tc(attn, denw, batch, Wr1, br1, Wr2, br2)
